# Initial kernel scaffold; baseline (speedup 1.0000x reference)
import jax
import jax.numpy as jnp
from jax.experimental import pallas as pl


def kernel(x, E):
    raise NotImplementedError("write your pallas kernel here")


# speedup vs baseline: 1.1016x; 1.1016x over previous
import functools

import jax
import jax.numpy as jnp
from jax import lax
from jax.experimental import pallas as pl
from jax.experimental.pallas import tpu as pltpu
from jax.experimental.pallas import tpu_sc as plsc

N_ROWS = 16384 * 100
D = 32
NC = 2
NS = 16
NW = NC * NS
B_PER_W = N_ROWS // NW
STREAM = 128
CHUNK = 1024
K = CHUNK // STREAM
N_CHUNKS = B_PER_W // CHUNK


def _sc_gather(x_flat, E):
    mesh = plsc.VectorSubcoreMesh(core_axis_name="c", subcore_axis_name="s")

    @functools.partial(
        pl.kernel,
        mesh=mesh,
        out_type=jax.ShapeDtypeStruct((N_ROWS, D), jnp.float32),
        compiler_params=pltpu.CompilerParams(use_tc_tiling_on_sc=False),
        scratch_types=[
            pltpu.VMEM((CHUNK,), jnp.int32),
            pltpu.VMEM((CHUNK, D), jnp.float32),
            pltpu.SemaphoreType.DMA,
        ],
    )
    def gather_kernel(idx_hbm, table_hbm, out_hbm, idx_v, rows_v, sem):
        wid = lax.axis_index("s") * NC + lax.axis_index("c")
        base = wid * B_PER_W

        def body(g, carry):
            off = base + g * CHUNK
            pltpu.sync_copy(idx_hbm.at[pl.ds(off, CHUNK)], idx_v)
            for j in range(K):
                pltpu.async_copy(
                    table_hbm.at[idx_v.at[pl.ds(j * STREAM, STREAM)]],
                    rows_v.at[pl.ds(j * STREAM, STREAM)],
                    sem,
                )
            for j in range(K):
                pltpu.make_async_copy(
                    table_hbm.at[idx_v.at[pl.ds(j * STREAM, STREAM)]],
                    rows_v.at[pl.ds(j * STREAM, STREAM)],
                    sem,
                ).wait()
            pltpu.sync_copy(rows_v, out_hbm.at[pl.ds(off, CHUNK)])
            return carry

        lax.fori_loop(0, N_CHUNKS, body, 0)

    return gather_kernel(x_flat, E)


def kernel(x, E):
    out = _sc_gather(x.reshape(-1), E)
    return out.reshape(x.shape[0], x.shape[1], D)


# speedup vs baseline: 1.1016x; 1.0000x over previous
import functools

import jax
import jax.numpy as jnp
from jax import lax
from jax.experimental import pallas as pl
from jax.experimental.pallas import tpu as pltpu
from jax.experimental.pallas import tpu_sc as plsc

N_ROWS = 16384 * 100
D = 32
NC = 2
NS = 16
NW = NC * NS
B_PER_W = N_ROWS // NW
STREAM = 1024
CHUNK = 1024
K = CHUNK // STREAM
N_CHUNKS = B_PER_W // CHUNK


def _sc_gather(x_flat, E):
    mesh = plsc.VectorSubcoreMesh(core_axis_name="c", subcore_axis_name="s")

    @functools.partial(
        pl.kernel,
        mesh=mesh,
        out_type=jax.ShapeDtypeStruct((N_ROWS, D), jnp.float32),
        compiler_params=pltpu.CompilerParams(use_tc_tiling_on_sc=False),
        scratch_types=[
            pltpu.VMEM((CHUNK,), jnp.int32),
            pltpu.VMEM((CHUNK, D), jnp.float32),
            pltpu.SemaphoreType.DMA,
        ],
    )
    def gather_kernel(idx_hbm, table_hbm, out_hbm, idx_v, rows_v, sem):
        wid = lax.axis_index("s") * NC + lax.axis_index("c")
        base = wid * B_PER_W

        def body(g, carry):
            off = base + g * CHUNK
            pltpu.sync_copy(idx_hbm.at[pl.ds(off, CHUNK)], idx_v)
            for j in range(K):
                pltpu.async_copy(
                    table_hbm.at[idx_v.at[pl.ds(j * STREAM, STREAM)]],
                    rows_v.at[pl.ds(j * STREAM, STREAM)],
                    sem,
                )
            for j in range(K):
                pltpu.make_async_copy(
                    table_hbm.at[idx_v.at[pl.ds(j * STREAM, STREAM)]],
                    rows_v.at[pl.ds(j * STREAM, STREAM)],
                    sem,
                ).wait()
            pltpu.sync_copy(rows_v, out_hbm.at[pl.ds(off, CHUNK)])
            return carry

        lax.fori_loop(0, N_CHUNKS, body, 0)

    return gather_kernel(x_flat, E)


def kernel(x, E):
    out = _sc_gather(x.reshape(-1), E)
    return out.reshape(x.shape[0], x.shape[1], D)


# speedup vs baseline: 3.1759x; 2.8829x over previous
import functools

import jax
import jax.numpy as jnp
from jax import lax
from jax.experimental import pallas as pl
from jax.experimental.pallas import tpu as pltpu
from jax.experimental.pallas import tpu_sc as plsc

B = 16384
T = 100
D = 32
NC = 2
NS = 16
NW = NC * NS
B_PER_W = B // NW
G = B_PER_W // 16


def _sc_gather(xt, E):
    mesh = plsc.VectorSubcoreMesh(core_axis_name="c", subcore_axis_name="s")

    @functools.partial(
        pl.kernel,
        mesh=mesh,
        out_type=jax.ShapeDtypeStruct((T, D, B), jnp.float32),
        compiler_params=pltpu.CompilerParams(
            use_tc_tiling_on_sc=False, needs_layout_passes=False
        ),
        scratch_types=[
            pltpu.VMEM((B_PER_W,), jnp.int32),
            pltpu.VMEM((B_PER_W, D), jnp.float32),
            pltpu.VMEM((D * B_PER_W,), jnp.float32),
            pltpu.SemaphoreType.DMA,
            pltpu.SemaphoreType.DMA,
        ],
    )
    def gather_kernel(xt_hbm, table_hbm, out_hbm, idx_v, rows_v, tr_v,
                      gsem, wsem):
        wid = lax.axis_index("s") * NC + lax.axis_index("c")
        b0 = wid * B_PER_W
        lane = lax.iota(jnp.int32, 16)

        def body(t, carry):
            pltpu.sync_copy(xt_hbm.at[t, pl.ds(b0, B_PER_W)], idx_v)
            pltpu.async_copy(table_hbm.at[idx_v], rows_v, gsem).wait()

            def tr_body(d, c):
                d_vec = jnp.full((16,), 0, jnp.int32) + d
                for g in range(G):
                    vals = plsc.load_gather(rows_v, [g * 16 + lane, d_vec])
                    tr_v[pl.ds(d * B_PER_W + g * 16, 16)] = vals
                pltpu.async_copy(
                    tr_v.at[pl.ds(d * B_PER_W, B_PER_W)],
                    out_hbm.at[t, d, pl.ds(b0, B_PER_W)],
                    wsem,
                )
                return c

            lax.fori_loop(0, D, tr_body, 0)

            def drain_body(d, c):
                pltpu.make_async_copy(
                    tr_v.at[pl.ds(d * B_PER_W, B_PER_W)],
                    out_hbm.at[t, d, pl.ds(b0, B_PER_W)],
                    wsem,
                ).wait()
                return c

            lax.fori_loop(0, D, drain_body, 0)
            return carry

        lax.fori_loop(0, T, body, 0)

    return gather_kernel(xt, E)


def kernel(x, E):
    out = _sc_gather(x.T, E)
    return jnp.transpose(out, (2, 0, 1))


# speedup vs baseline: 3.9058x; 1.2298x over previous
import functools

import jax
import jax.numpy as jnp
from jax import lax
from jax.experimental import pallas as pl
from jax.experimental.pallas import tpu as pltpu
from jax.experimental.pallas import tpu_sc as plsc

B = 16384
T = 100
D = 32
NC = 2
NS = 16
NW = NC * NS
BW = B // NW
CT = BW // 128
GT = D // 8


def _sc_gather(xt, E):
    mesh = plsc.VectorSubcoreMesh(core_axis_name="c", subcore_axis_name="s")

    @functools.partial(
        pl.kernel,
        mesh=mesh,
        out_type=jax.ShapeDtypeStruct((T, GT, B // 128, 8, 128), jnp.float32),
        compiler_params=pltpu.CompilerParams(
            use_tc_tiling_on_sc=False, needs_layout_passes=False
        ),
        scratch_types=[
            pltpu.VMEM((T, BW), jnp.int32),
            pltpu.VMEM((BW, D), jnp.float32),
            pltpu.VMEM((BW, D), jnp.float32),
            pltpu.VMEM((GT, CT, 8, 128), jnp.float32),
            pltpu.VMEM((GT, CT, 8, 128), jnp.float32),
            pltpu.SemaphoreType.DMA,
            pltpu.SemaphoreType.DMA,
            pltpu.SemaphoreType.DMA,
            pltpu.SemaphoreType.DMA,
        ],
    )
    def gather_kernel(xt_hbm, table_hbm, out_hbm, idx_v, rows_a, rows_b,
                      tr_a, tr_b, gsem_a, gsem_b, wsem_a, wsem_b):
        wid = lax.axis_index("s") * NC + lax.axis_index("c")
        b0 = wid * BW
        lane = lax.iota(jnp.int32, 16)

        pltpu.sync_copy(xt_hbm.at[:, pl.ds(b0, BW)], idx_v)

        def issue_gather(t, rows, gsem):
            pltpu.async_copy(table_hbm.at[idx_v.at[t]], rows, gsem)

        def wait_gather(t, rows, gsem):
            pltpu.make_async_copy(table_hbm.at[idx_v.at[t]], rows, gsem).wait()

        def transpose(rows, tr):
            def k_body(k, carry):
                g = k // CT
                c = k % CT
                d_base = g * 8
                l_base = c * 128
                for r in range(8):
                    d_vec = jnp.full((16,), 0, jnp.int32) + (d_base + r)
                    for lh in range(8):
                        b_vec = lane + (l_base + lh * 16)
                        vals = plsc.load_gather(rows, [b_vec, d_vec])
                        tr[g, c, r, pl.ds(lh * 16, 16)] = vals
                return carry

            lax.fori_loop(0, GT * CT, k_body, 0)

        def issue_writes(t, tr, wsem):
            for g in range(GT):
                pltpu.async_copy(
                    tr.at[g], out_hbm.at[t, g, pl.ds(wid * CT, CT)], wsem
                )

        def drain_writes(t, tr, wsem):
            for g in range(GT):
                pltpu.make_async_copy(
                    tr.at[g], out_hbm.at[t, g, pl.ds(wid * CT, CT)], wsem
                ).wait()

        def step(t, rows_cur, tr_cur, rows_nxt, gsem_cur, gsem_nxt, wsem_cur):
            @pl.when(t + 1 < T)
            def _():
                issue_gather(t + 1, rows_nxt, gsem_nxt)

            wait_gather(t, rows_cur, gsem_cur)

            @pl.when(t >= 2)
            def _():
                drain_writes(t - 2, tr_cur, wsem_cur)

            transpose(rows_cur, tr_cur)
            issue_writes(t, tr_cur, wsem_cur)

        issue_gather(0, rows_a, gsem_a)

        def pair_body(i, carry):
            t0 = 2 * i
            step(t0, rows_a, tr_a, rows_b, gsem_a, gsem_b, wsem_a)
            step(t0 + 1, rows_b, tr_b, rows_a, gsem_b, gsem_a, wsem_b)
            return carry

        lax.fori_loop(0, T // 2, pair_body, 0)
        drain_writes(T - 2, tr_a, wsem_a)
        drain_writes(T - 1, tr_b, wsem_b)

    return gather_kernel(xt, E)


def kernel(x, E):
    out5 = _sc_gather(x.T, E)
    out = jnp.transpose(out5, (2, 4, 0, 1, 3)).reshape(B, T, D)
    return out


# speedup vs baseline: 4.3356x; 1.1100x over previous
import functools

import jax
import jax.numpy as jnp
from jax import lax
from jax.experimental import pallas as pl
from jax.experimental.pallas import tpu as pltpu
from jax.experimental.pallas import tpu_sc as plsc

B = 16384
T = 100
D = 32
NC = 2
NS = 16
NW = NC * NS
BW = B // NW
CT = BW // 128
GT = D // 8
MB = BW // 16


def _sc_gather(xt, E):
    mesh = plsc.VectorSubcoreMesh(core_axis_name="c", subcore_axis_name="s")

    @functools.partial(
        pl.kernel,
        mesh=mesh,
        out_type=jax.ShapeDtypeStruct((T, GT, (B // 128) * 1024), jnp.float32),
        compiler_params=pltpu.CompilerParams(
            use_tc_tiling_on_sc=False, needs_layout_passes=False
        ),
        scratch_types=[
            pltpu.VMEM((T, BW), jnp.int32),
            pltpu.VMEM((BW, D), jnp.float32),
            pltpu.VMEM((BW, D), jnp.float32),
            pltpu.VMEM((GT * CT * 8 * 128,), jnp.float32),
            pltpu.VMEM((GT * CT * 8 * 128,), jnp.float32),
            pltpu.VMEM((D, 16), jnp.int32),
            pltpu.VMEM((D, 16), jnp.int32),
            pltpu.SemaphoreType.DMA,
            pltpu.SemaphoreType.DMA,
            pltpu.SemaphoreType.DMA,
            pltpu.SemaphoreType.DMA,
        ],
    )
    def gather_kernel(xt_hbm, table_hbm, out_hbm, idx_v, rows_a, rows_b,
                      tr_a, tr_b, dtab, wtab, gsem_a, gsem_b, wsem_a, wsem_b):
        wid = lax.axis_index("s") * NC + lax.axis_index("c")
        b0 = wid * BW
        lane = lax.iota(jnp.int32, 16)

        def tab_body(j, carry):
            d_vec = (j + lane) & 31
            dtab[j, :] = d_vec
            wtab[j, :] = ((d_vec >> 3) * 4096 + (d_vec & 7) * 128) + lane
            return carry

        lax.fori_loop(0, D, tab_body, 0)

        pltpu.sync_copy(xt_hbm.at[:, pl.ds(b0, BW)], idx_v)

        def issue_gather(t, rows, gsem):
            pltpu.async_copy(table_hbm.at[idx_v.at[t]], rows, gsem)

        def wait_gather(t, rows, gsem):
            pltpu.make_async_copy(table_hbm.at[idx_v.at[t]], rows, gsem).wait()

        def transpose(rows, tr):
            def m_body(m, carry):
                b_vec = lane + m * 16
                c = m // 8
                dst_base = c * 1024 + (m % 8) * 16
                for j in range(D):
                    d_vec = dtab[j, :]
                    vals = plsc.load_gather(rows, [b_vec, d_vec])
                    plsc.store_scatter(tr, [wtab[j, :] + dst_base], vals)
                return carry

            lax.fori_loop(0, MB, m_body, 0)

        def issue_writes(t, tr, wsem):
            for g in range(GT):
                pltpu.async_copy(
                    tr.at[pl.ds(g * CT * 1024, CT * 1024)],
                    out_hbm.at[t, g, pl.ds(wid * CT * 1024, CT * 1024)],
                    wsem,
                )

        def drain_writes(t, tr, wsem):
            for g in range(GT):
                pltpu.make_async_copy(
                    tr.at[pl.ds(g * CT * 1024, CT * 1024)],
                    out_hbm.at[t, g, pl.ds(wid * CT * 1024, CT * 1024)],
                    wsem,
                ).wait()

        def step(t, rows_cur, tr_cur, rows_nxt, gsem_cur, gsem_nxt, wsem_cur):
            @pl.when(t + 1 < T)
            def _():
                issue_gather(t + 1, rows_nxt, gsem_nxt)

            wait_gather(t, rows_cur, gsem_cur)

            @pl.when(t >= 2)
            def _():
                drain_writes(t - 2, tr_cur, wsem_cur)

            transpose(rows_cur, tr_cur)
            issue_writes(t, tr_cur, wsem_cur)

        issue_gather(0, rows_a, gsem_a)

        def pair_body(i, carry):
            t0 = 2 * i
            step(t0, rows_a, tr_a, rows_b, gsem_a, gsem_b, wsem_a)
            step(t0 + 1, rows_b, tr_b, rows_a, gsem_b, gsem_a, wsem_b)
            return carry

        lax.fori_loop(0, T // 2, pair_body, 0)
        drain_writes(T - 2, tr_a, wsem_a)
        drain_writes(T - 1, tr_b, wsem_b)

    return gather_kernel(xt, E)


def kernel(x, E):
    out5 = _sc_gather(x.T, E)
    out5 = out5.reshape(T, GT, B // 128, 8, 128)
    out = jnp.transpose(out5, (2, 4, 0, 1, 3)).reshape(B, T, D)
    return out


# speedup vs baseline: 7.2535x; 1.6730x over previous
import functools

import jax
import jax.numpy as jnp
from jax import lax
from jax.experimental import pallas as pl
from jax.experimental.pallas import tpu as pltpu
from jax.experimental.pallas import tpu_sc as plsc

B = 16384
T = 100
D = 32
NC = 2
NS = 16
NW = NC * NS
BW = B // NW
CT = BW // 128
GT = D // 8
MB = BW // 16


def _sc_gather(xt, E):
    mesh = plsc.VectorSubcoreMesh(core_axis_name="c", subcore_axis_name="s")

    @functools.partial(
        pl.kernel,
        mesh=mesh,
        out_type=jax.ShapeDtypeStruct((T, GT, (B // 128) * 1024), jnp.float32),
        compiler_params=pltpu.CompilerParams(
            use_tc_tiling_on_sc=False, needs_layout_passes=False
        ),
        scratch_types=[
            pltpu.VMEM((T, BW), jnp.int32),
            pltpu.VMEM((BW, D), jnp.float32),
            pltpu.VMEM((BW, D), jnp.float32),
            pltpu.VMEM((GT * CT * 8 * 128,), jnp.float32),
            pltpu.VMEM((GT * CT * 8 * 128,), jnp.float32),
            pltpu.VMEM((D, 16), jnp.int32),
            pltpu.VMEM((D, 16), jnp.int32),
            pltpu.SemaphoreType.DMA,
            pltpu.SemaphoreType.DMA,
            pltpu.SemaphoreType.DMA,
            pltpu.SemaphoreType.DMA,
        ],
    )
    def gather_kernel(xt_hbm, table_hbm, out_hbm, idx_v, rows_a, rows_b,
                      tr_a, tr_b, dtab, wtab, gsem_a, gsem_b, wsem_a, wsem_b):
        wid = lax.axis_index("s") * NC + lax.axis_index("c")
        b0 = wid * BW
        lane = lax.iota(jnp.int32, 16)

        def tab_body(j, carry):
            d_vec = (j + lane) & 31
            dtab[j, :] = d_vec
            wtab[j, :] = ((d_vec >> 3) * 4096 + (d_vec & 7) * 128) + lane
            return carry

        lax.fori_loop(0, D, tab_body, 0)

        pltpu.sync_copy(xt_hbm.at[:, pl.ds(b0, BW)], idx_v)

        def issue_gather(t, rows, gsem):
            pltpu.async_copy(table_hbm.at[idx_v.at[t]], rows, gsem)

        def wait_gather(t, rows, gsem):
            pltpu.make_async_copy(table_hbm.at[idx_v.at[t]], rows, gsem).wait()

        def transpose(rows, tr):
            def j_body(j, carry):
                d_vec = dtab[j, :]
                w_vec = wtab[j, :]
                for m in range(MB):
                    b_vec = lane + m * 16
                    dst_base = (m // 8) * 1024 + (m % 8) * 16
                    vals = plsc.load_gather(rows, [b_vec, d_vec])
                    plsc.store_scatter(tr, [w_vec + dst_base], vals)
                return carry

            lax.fori_loop(0, D, j_body, 0)

        def issue_writes(t, tr, wsem):
            for g in range(GT):
                pltpu.async_copy(
                    tr.at[pl.ds(g * CT * 1024, CT * 1024)],
                    out_hbm.at[t, g, pl.ds(wid * CT * 1024, CT * 1024)],
                    wsem,
                )

        def drain_writes(t, tr, wsem):
            for g in range(GT):
                pltpu.make_async_copy(
                    tr.at[pl.ds(g * CT * 1024, CT * 1024)],
                    out_hbm.at[t, g, pl.ds(wid * CT * 1024, CT * 1024)],
                    wsem,
                ).wait()

        def step(t, rows_cur, tr_cur, rows_nxt, gsem_cur, gsem_nxt, wsem_cur):
            @pl.when(t + 1 < T)
            def _():
                issue_gather(t + 1, rows_nxt, gsem_nxt)

            wait_gather(t, rows_cur, gsem_cur)

            @pl.when(t >= 2)
            def _():
                drain_writes(t - 2, tr_cur, wsem_cur)

            transpose(rows_cur, tr_cur)
            issue_writes(t, tr_cur, wsem_cur)

        issue_gather(0, rows_a, gsem_a)

        def pair_body(i, carry):
            t0 = 2 * i
            step(t0, rows_a, tr_a, rows_b, gsem_a, gsem_b, wsem_a)
            step(t0 + 1, rows_b, tr_b, rows_a, gsem_b, gsem_a, wsem_b)
            return carry

        lax.fori_loop(0, T // 2, pair_body, 0)
        drain_writes(T - 2, tr_a, wsem_a)
        drain_writes(T - 1, tr_b, wsem_b)

    return gather_kernel(xt, E)


def kernel(x, E):
    out5 = _sc_gather(x.T, E)
    out5 = out5.reshape(T, GT, B // 128, 8, 128)
    out = jnp.transpose(out5, (2, 4, 0, 1, 3)).reshape(B, T, D)
    return out


# speedup vs baseline: 8.9638x; 1.2358x over previous
import functools

import jax
import jax.numpy as jnp
from jax import lax
from jax.experimental import pallas as pl
from jax.experimental.pallas import tpu as pltpu
from jax.experimental.pallas import tpu_sc as plsc

B = 16384
T = 100
D = 32
NC = 2
NS = 16
NW = NC * NS
BW = B // NW
CT = BW // 128
GT = D // 8
MB = BW // 16


def _sc_gather(xt, E):
    mesh = plsc.VectorSubcoreMesh(core_axis_name="c", subcore_axis_name="s")

    @functools.partial(
        pl.kernel,
        mesh=mesh,
        out_type=jax.ShapeDtypeStruct((T, GT, (B // 128) * 1024), jnp.float32),
        compiler_params=pltpu.CompilerParams(
            use_tc_tiling_on_sc=False, needs_layout_passes=False
        ),
        scratch_types=[
            pltpu.VMEM((T, BW), jnp.int32),
            pltpu.VMEM((BW, D), jnp.float32),
            pltpu.VMEM((BW, D), jnp.float32),
            pltpu.VMEM((GT * CT * 8 * 128,), jnp.float32),
            pltpu.VMEM((GT * CT * 8 * 128,), jnp.float32),
            pltpu.VMEM((D, 16), jnp.int32),
            pltpu.VMEM((D, 16), jnp.int32),
            pltpu.SemaphoreType.DMA,
            pltpu.SemaphoreType.DMA,
            pltpu.SemaphoreType.DMA,
            pltpu.SemaphoreType.DMA,
        ],
    )
    def gather_kernel(xt_hbm, table_hbm, out_hbm, idx_v, rows_a, rows_b,
                      tr_a, tr_b, dtab, wtab, gsem_a, gsem_b, wsem_a, wsem_b):
        wid = lax.axis_index("s") * NC + lax.axis_index("c")
        b0 = wid * BW
        lane = lax.iota(jnp.int32, 16)

        def tab_body(j, carry):
            d_vec = (j + lane) & 31
            dtab[j, :] = d_vec
            wtab[j, :] = ((d_vec >> 3) * 4096 + (d_vec & 7) * 128) + lane
            return carry

        lax.fori_loop(0, D, tab_body, 0)

        pltpu.sync_copy(xt_hbm.at[:, pl.ds(b0, BW)], idx_v)

        def issue_gather(t, rows, gsem):
            pltpu.async_copy(table_hbm.at[idx_v.at[t]], rows, gsem)

        def wait_gather(t, rows, gsem):
            pltpu.make_async_copy(table_hbm.at[idx_v.at[t]], rows, gsem).wait()

        def transpose(rows, tr):
            def j_body(j, carry):
                d_vec0 = dtab[j, :]
                w_vec0 = wtab[j, :]
                d_vec1 = dtab[j + D // 2, :]
                w_vec1 = wtab[j + D // 2, :]
                for m in range(MB):
                    b_vec = lane + m * 16
                    dst_base = (m // 8) * 1024 + (m % 8) * 16
                    vals0 = plsc.load_gather(rows, [b_vec, d_vec0])
                    vals1 = plsc.load_gather(rows, [b_vec, d_vec1])
                    plsc.store_scatter(tr, [w_vec0 + dst_base], vals0)
                    plsc.store_scatter(tr, [w_vec1 + dst_base], vals1)
                return carry

            lax.fori_loop(0, D // 2, j_body, 0)

        def issue_writes(t, tr, wsem):
            for g in range(GT):
                pltpu.async_copy(
                    tr.at[pl.ds(g * CT * 1024, CT * 1024)],
                    out_hbm.at[t, g, pl.ds(wid * CT * 1024, CT * 1024)],
                    wsem,
                )

        def drain_writes(t, tr, wsem):
            for g in range(GT):
                pltpu.make_async_copy(
                    tr.at[pl.ds(g * CT * 1024, CT * 1024)],
                    out_hbm.at[t, g, pl.ds(wid * CT * 1024, CT * 1024)],
                    wsem,
                ).wait()

        def step(t, rows_cur, tr_cur, rows_nxt, gsem_cur, gsem_nxt, wsem_cur):
            @pl.when(t + 1 < T)
            def _():
                issue_gather(t + 1, rows_nxt, gsem_nxt)

            wait_gather(t, rows_cur, gsem_cur)

            @pl.when(t >= 2)
            def _():
                drain_writes(t - 2, tr_cur, wsem_cur)

            transpose(rows_cur, tr_cur)
            issue_writes(t, tr_cur, wsem_cur)

        issue_gather(0, rows_a, gsem_a)

        def pair_body(i, carry):
            t0 = 2 * i
            step(t0, rows_a, tr_a, rows_b, gsem_a, gsem_b, wsem_a)
            step(t0 + 1, rows_b, tr_b, rows_a, gsem_b, gsem_a, wsem_b)
            return carry

        lax.fori_loop(0, T // 2, pair_body, 0)
        drain_writes(T - 2, tr_a, wsem_a)
        drain_writes(T - 1, tr_b, wsem_b)

    return gather_kernel(xt, E)


def kernel(x, E):
    out5 = _sc_gather(x.T, E)
    out5 = out5.reshape(T, GT, B // 128, 8, 128)
    out = jnp.transpose(out5, (2, 4, 0, 1, 3)).reshape(B, T, D)
    return out


# speedup vs baseline: 9.2918x; 1.0366x over previous
import functools

import jax
import jax.numpy as jnp
from jax import lax
from jax.experimental import pallas as pl
from jax.experimental.pallas import tpu as pltpu
from jax.experimental.pallas import tpu_sc as plsc

B = 16384
T = 100
D = 32
NC = 2
NS = 16
NW = NC * NS
BW = B // NW
CT = BW // 128
GT = D // 8
MB = BW // 16


def _sc_gather(xt, E):
    mesh = plsc.VectorSubcoreMesh(core_axis_name="c", subcore_axis_name="s")

    @functools.partial(
        pl.kernel,
        mesh=mesh,
        out_type=jax.ShapeDtypeStruct((T, GT, (B // 128) * 1024), jnp.float32),
        compiler_params=pltpu.CompilerParams(
            use_tc_tiling_on_sc=False, needs_layout_passes=False
        ),
        scratch_types=[
            pltpu.VMEM((T, BW), jnp.int32),
            pltpu.VMEM((BW, D), jnp.float32),
            pltpu.VMEM((BW, D), jnp.float32),
            pltpu.VMEM((GT * CT * 8 * 128,), jnp.float32),
            pltpu.VMEM((GT * CT * 8 * 128,), jnp.float32),
            pltpu.VMEM((D, 16), jnp.int32),
            pltpu.VMEM((D, 16), jnp.int32),
            pltpu.SemaphoreType.DMA,
            pltpu.SemaphoreType.DMA,
            pltpu.SemaphoreType.DMA,
            pltpu.SemaphoreType.DMA,
        ],
    )
    def gather_kernel(xt_hbm, table_hbm, out_hbm, idx_v, rows_a, rows_b,
                      tr_a, tr_b, dtab, wtab, gsem_a, gsem_b, wsem_a, wsem_b):
        wid = lax.axis_index("s") * NC + lax.axis_index("c")
        b0 = wid * BW
        lane = lax.iota(jnp.int32, 16)

        def tab_body(j, carry):
            d_vec = (j + lane) & 31
            dtab[j, :] = d_vec
            wtab[j, :] = ((d_vec >> 3) * 4096 + (d_vec & 7) * 128) + lane
            return carry

        lax.fori_loop(0, D, tab_body, 0)

        pltpu.sync_copy(xt_hbm.at[:, pl.ds(b0, BW)], idx_v)

        def issue_gather(t, rows, gsem):
            pltpu.async_copy(table_hbm.at[idx_v.at[t]], rows, gsem)

        def wait_gather(t, rows, gsem):
            pltpu.make_async_copy(table_hbm.at[idx_v.at[t]], rows, gsem).wait()

        def transpose(rows, tr):
            def j_body(j, carry):
                d_vecs = [dtab[j + k * (D // 4), :] for k in range(4)]
                w_vecs = [wtab[j + k * (D // 4), :] for k in range(4)]
                for m in range(MB):
                    b_vec = lane + m * 16
                    dst_base = (m // 8) * 1024 + (m % 8) * 16
                    vals = [
                        plsc.load_gather(rows, [b_vec, d_vecs[k]])
                        for k in range(4)
                    ]
                    for k in range(4):
                        plsc.store_scatter(tr, [w_vecs[k] + dst_base], vals[k])
                return carry

            lax.fori_loop(0, D // 4, j_body, 0)

        def issue_writes(t, tr, wsem):
            for g in range(GT):
                pltpu.async_copy(
                    tr.at[pl.ds(g * CT * 1024, CT * 1024)],
                    out_hbm.at[t, g, pl.ds(wid * CT * 1024, CT * 1024)],
                    wsem,
                )

        def drain_writes(t, tr, wsem):
            for g in range(GT):
                pltpu.make_async_copy(
                    tr.at[pl.ds(g * CT * 1024, CT * 1024)],
                    out_hbm.at[t, g, pl.ds(wid * CT * 1024, CT * 1024)],
                    wsem,
                ).wait()

        def step(t, rows_cur, tr_cur, rows_nxt, gsem_cur, gsem_nxt, wsem_cur):
            @pl.when(t + 1 < T)
            def _():
                issue_gather(t + 1, rows_nxt, gsem_nxt)

            wait_gather(t, rows_cur, gsem_cur)

            @pl.when(t >= 2)
            def _():
                drain_writes(t - 2, tr_cur, wsem_cur)

            transpose(rows_cur, tr_cur)
            issue_writes(t, tr_cur, wsem_cur)

        issue_gather(0, rows_a, gsem_a)

        def pair_body(i, carry):
            t0 = 2 * i
            step(t0, rows_a, tr_a, rows_b, gsem_a, gsem_b, wsem_a)
            step(t0 + 1, rows_b, tr_b, rows_a, gsem_b, gsem_a, wsem_b)
            return carry

        lax.fori_loop(0, T // 2, pair_body, 0)
        drain_writes(T - 2, tr_a, wsem_a)
        drain_writes(T - 1, tr_b, wsem_b)

    return gather_kernel(xt, E)


def kernel(x, E):
    out5 = _sc_gather(x.T, E)
    out5 = out5.reshape(T, GT, B // 128, 8, 128)
    out = jnp.transpose(out5, (2, 4, 0, 1, 3)).reshape(B, T, D)
    return out


# speedup vs baseline: 10.2467x; 1.1028x over previous
import functools

import jax
import jax.numpy as jnp
from jax import lax
from jax.experimental import pallas as pl
from jax.experimental.pallas import tpu as pltpu
from jax.experimental.pallas import tpu_sc as plsc

B = 16384
T = 100
D = 32
NC = 2
NS = 16
NW = NC * NS
BW = B // NW
CT = BW // 128
GT = D // 8
MB = BW // 16


def _sc_gather(xt, E):
    mesh = plsc.VectorSubcoreMesh(core_axis_name="c", subcore_axis_name="s")

    @functools.partial(
        pl.kernel,
        mesh=mesh,
        out_type=jax.ShapeDtypeStruct((T, GT, (B // 128) * 1024), jnp.float32),
        compiler_params=pltpu.CompilerParams(
            use_tc_tiling_on_sc=False, needs_layout_passes=False
        ),
        scratch_types=[
            pltpu.VMEM((T, BW), jnp.int32),
            pltpu.VMEM((BW, D), jnp.float32),
            pltpu.VMEM((BW, D), jnp.float32),
            pltpu.VMEM((GT * CT * 8 * 128,), jnp.float32),
            pltpu.VMEM((GT * CT * 8 * 128,), jnp.float32),
            pltpu.VMEM((D, 16), jnp.int32),
            pltpu.VMEM((D, 16), jnp.int32),
            pltpu.SemaphoreType.DMA,
            pltpu.SemaphoreType.DMA,
            pltpu.SemaphoreType.DMA,
            pltpu.SemaphoreType.DMA,
        ],
    )
    def gather_kernel(xt_hbm, table_hbm, out_hbm, idx_v, rows_a, rows_b,
                      tr_a, tr_b, dtab, wtab, gsem_a, gsem_b, wsem_a, wsem_b):
        wid = lax.axis_index("s") * NC + lax.axis_index("c")
        b0 = wid * BW
        lane = lax.iota(jnp.int32, 16)

        def tab_body(j, carry):
            d_vec = (j + lane) & 31
            dtab[j, :] = d_vec
            wtab[j, :] = ((d_vec >> 3) * 4096 + (d_vec & 7) * 128) + lane
            return carry

        lax.fori_loop(0, D, tab_body, 0)

        pltpu.sync_copy(xt_hbm.at[:, pl.ds(b0, BW)], idx_v)

        def issue_gather(t, rows, gsem):
            pltpu.async_copy(table_hbm.at[idx_v.at[t]], rows, gsem)

        def wait_gather(t, rows, gsem):
            pltpu.make_async_copy(table_hbm.at[idx_v.at[t]], rows, gsem).wait()

        def transpose(rows, tr):
            NI = 8
            def j_body(j, carry):
                d_vecs = [dtab[j + k * (D // NI), :] for k in range(NI)]
                w_vecs = [wtab[j + k * (D // NI), :] for k in range(NI)]
                for m in range(MB):
                    b_vec = lane + m * 16
                    dst_base = (m // 8) * 1024 + (m % 8) * 16
                    vals = [
                        plsc.load_gather(rows, [b_vec, d_vecs[k]])
                        for k in range(NI)
                    ]
                    for k in range(NI):
                        plsc.store_scatter(tr, [w_vecs[k] + dst_base], vals[k])
                return carry

            lax.fori_loop(0, D // NI, j_body, 0)

        def issue_writes(t, tr, wsem):
            for g in range(GT):
                pltpu.async_copy(
                    tr.at[pl.ds(g * CT * 1024, CT * 1024)],
                    out_hbm.at[t, g, pl.ds(wid * CT * 1024, CT * 1024)],
                    wsem,
                )

        def drain_writes(t, tr, wsem):
            for g in range(GT):
                pltpu.make_async_copy(
                    tr.at[pl.ds(g * CT * 1024, CT * 1024)],
                    out_hbm.at[t, g, pl.ds(wid * CT * 1024, CT * 1024)],
                    wsem,
                ).wait()

        def step(t, rows_cur, tr_cur, rows_nxt, gsem_cur, gsem_nxt, wsem_cur):
            @pl.when(t + 1 < T)
            def _():
                issue_gather(t + 1, rows_nxt, gsem_nxt)

            wait_gather(t, rows_cur, gsem_cur)

            @pl.when(t >= 2)
            def _():
                drain_writes(t - 2, tr_cur, wsem_cur)

            transpose(rows_cur, tr_cur)
            issue_writes(t, tr_cur, wsem_cur)

        issue_gather(0, rows_a, gsem_a)

        def pair_body(i, carry):
            t0 = 2 * i
            step(t0, rows_a, tr_a, rows_b, gsem_a, gsem_b, wsem_a)
            step(t0 + 1, rows_b, tr_b, rows_a, gsem_b, gsem_a, wsem_b)
            return carry

        lax.fori_loop(0, T // 2, pair_body, 0)
        drain_writes(T - 2, tr_a, wsem_a)
        drain_writes(T - 1, tr_b, wsem_b)

    return gather_kernel(xt, E)


def kernel(x, E):
    out5 = _sc_gather(x.T, E)
    out5 = out5.reshape(T, GT, B // 128, 8, 128)
    out = jnp.transpose(out5, (2, 4, 0, 1, 3)).reshape(B, T, D)
    return out
